# Initial kernel scaffold; baseline (speedup 1.0000x reference)
import jax
import jax.numpy as jnp
from jax.experimental import pallas as pl


def kernel(preds, tgt, tgt_pos, reward):
    raise NotImplementedError("write your pallas kernel here")


# speedup vs baseline: 3.1482x; 3.1482x over previous
import jax
import jax.numpy as jnp
from jax.experimental import pallas as pl

_TB = 64


def _loss_block_kernel(preds_ref, tgt_ref, reward_ref, out_ref):
    i = pl.program_id(0)
    x = preds_ref[...]
    tb, v = x.shape
    m = jnp.max(x, axis=1, keepdims=True)
    e = jnp.exp(x - m)
    s = jnp.sum(e, axis=1)
    tgt = tgt_ref[0, 0, :]
    cols = jax.lax.broadcasted_iota(jnp.int32, (tb, v), 1)
    sel = jnp.sum(jnp.where(cols == tgt[:, None], e, 0.0), axis=1)
    mask = (tgt > 0).astype(jnp.float32)
    partial = jnp.sum(sel / s * mask * reward_ref[0, 0, :])

    @pl.when(i == 0)
    def _init():
        out_ref[...] = jnp.zeros_like(out_ref)

    out_ref[...] += jnp.full(out_ref.shape, -partial, out_ref.dtype)


def kernel(preds, tgt, tgt_pos, reward):
    b, s, v = preds.shape
    n = b * s
    nt = n // _TB
    preds2 = preds.reshape(n, v)
    tgt3 = tgt.reshape(nt, 1, _TB)
    reward3 = reward.reshape(nt, 1, _TB)

    out = pl.pallas_call(
        _loss_block_kernel,
        grid=(nt,),
        in_specs=[
            pl.BlockSpec((_TB, v), lambda i: (i, 0)),
            pl.BlockSpec((1, 1, _TB), lambda i: (i, 0, 0)),
            pl.BlockSpec((1, 1, _TB), lambda i: (i, 0, 0)),
        ],
        out_specs=pl.BlockSpec((1, 1), lambda i: (0, 0)),
        out_shape=jax.ShapeDtypeStruct((1, 1), jnp.float32),
    )(preds2, tgt3, reward3)
    return out[0, 0]
